# baseline (speedup 1.0000x reference)
import functools
import numpy as np
import jax
import jax.numpy as jnp
from jax import lax
from jax.experimental import pallas as pl
from jax.experimental.pallas import tpu as pltpu
from jax.experimental.pallas import tpu_sc as plsc

HID = 256
NL = 3
DECAY = 0.9
EPS_TAU = 1e-08
EPS_LN = 1e-05
LN_DECAY = float(np.log(DECAY))

NC = 2
NS = 16
LANES = 16
HHID = HID // 2
CEDGE = 80


def _mesh():
    return plsc.VectorSubcoreMesh(core_axis_name="c", subcore_axis_name="s")


def _degwp_body(dst_hbm, pw_hbm, scale_hbm, deg_hbm, wp_hbm,
                dst0, dst1, pw0, pw1, wq16_v, ones_v, scale_v, deg_sh, sidx,
                *, E, NP, C, EPT, RPT):
    c = lax.axis_index("c")
    s = lax.axis_index("s")
    NCH = EPT // C
    PW = 2 * C

    def fill(val, i, _):
        for k in range(HHID // LANES):
            ones_v[i, pl.ds(k * LANES, LANES)] = jnp.full((LANES,), val,
                                                          jnp.float32)
        return 0
    lax.fori_loop(0, C, functools.partial(fill, 0.0), 0)
    zbase = s * RPT
    for t in range(RPT // C):
        pltpu.sync_copy(ones_v, deg_sh.at[pl.ds(zbase + t * C, C)])
    if RPT % C:
        pltpu.sync_copy(ones_v.at[pl.ds(0, RPT % C)],
                        deg_sh.at[pl.ds(zbase + (RPT // C) * C, RPT % C)])
    lax.fori_loop(0, C, functools.partial(fill, 1.0), 0)
    pltpu.sync_copy(scale_hbm, scale_v)
    plsc.subcore_barrier()

    g0 = s * NCH
    gend = g0 + NCH

    def issue_idx(ci, dst_v, pw_v):
        pltpu.async_copy(dst_hbm.at[pl.ds(ci * C, C)], dst_v, sidx)
        pltpu.async_copy(pw_hbm.at[pl.ds(ci * PW, PW)], pw_v, sidx)

    def wait_idx(ci, dst_v, pw_v):
        pltpu.make_async_copy(dst_hbm.at[pl.ds(ci * C, C)], dst_v,
                              sidx).wait()
        pltpu.make_async_copy(pw_hbm.at[pl.ds(ci * PW, PW)], pw_v,
                              sidx).wait()

    def wq(ci, pw_v):
        @pl.when(c == 0)
        def _():
            for j in range(C // LANES):
                pv = pw_v[pl.ds(j * LANES, LANES)]
                mv = pw_v[pl.ds(C + j * LANES, LANES)]
                dist = pv * jnp.exp(mv * LN_DECAY)
                w = jnp.exp(dist * scale_v[...])
                for kk in range(LANES):
                    e = j * LANES + kk
                    wq16_v[pl.ds(e * LANES, LANES)] = (
                        jnp.broadcast_to(w[kk], (LANES,)))
            pltpu.sync_copy(wq16_v,
                            wp_hbm.at[pl.ds(ci * C * LANES, C * LANES)])

    bufs = ((dst0, pw0), (dst1, pw1))
    pltpu.sync_copy(dst_hbm.at[pl.ds(g0 * C, C)], dst0)
    pltpu.sync_copy(pw_hbm.at[pl.ds(g0 * PW, PW)], pw0)
    issue_idx(g0 + 1, dst1, pw1)

    def sub(ci, A, B):
        dstA, pwA = A
        dstB, pwB = B
        wq(ci, pwA)
        wait_idx(ci + 1, dstB, pwB)

        @pl.when(ci + 2 < gend)
        def _():
            issue_idx(ci + 2, dstA, pwA)
        pltpu.sync_copy(ones_v, deg_sh.at[dstA], add=True)

    def body(i, _):
        ci0 = g0 + i * 2
        sub(ci0, bufs[0], bufs[1])
        sub(ci0 + 1, bufs[1], bufs[0])
        return 0
    lax.fori_loop(0, (NCH - 1) // 2, body, 0)

    lb = (NCH - 1) % 2
    dstL, pwL = bufs[lb]
    wq(gend - 1, pwL)
    pltpu.sync_copy(ones_v, deg_sh.at[dstL], add=True)

    plsc.subcore_barrier()
    pltpu.sync_copy(deg_sh.at[pl.ds(s * RPT, RPT)],
                    deg_hbm.at[pl.ds(c * NP + s * RPT, RPT)])


def _degwp(dst, pw, scale16, N):
    E = dst.shape[0]
    NP = ((N + NS * 8 - 1) // (NS * 8)) * (NS * 8)
    C = CEDGE
    EPT = E // NS
    RPT = NP // NS
    assert (EPT // C) % 2 == 1 and EPT % C == 0
    body = functools.partial(_degwp_body, E=E, NP=NP, C=C, EPT=EPT, RPT=RPT)
    f = pl.kernel(
        body,
        out_type=(jax.ShapeDtypeStruct((NC * NP, HHID), jnp.float32),
                  jax.ShapeDtypeStruct((E * LANES,), jnp.float32)),
        mesh=_mesh(),
        scratch_types=[
            pltpu.VMEM((C,), jnp.int32),
            pltpu.VMEM((C,), jnp.int32),
            pltpu.VMEM((2 * C,), jnp.float32),
            pltpu.VMEM((2 * C,), jnp.float32),
            pltpu.VMEM((C * LANES,), jnp.float32),
            pltpu.VMEM((C, HHID), jnp.float32),
            pltpu.VMEM((LANES,), jnp.float32),
            pltpu.VMEM_SHARED((NP, HHID), jnp.float32),
            pltpu.SemaphoreType.DMA,
        ],
    )
    return f(dst, pw, scale16) + (NP,)


def _msg_body(h_hbm, sd_hbm, wp_hbm, msg_hbm,
              sd0, sd1, wp0, wp1, src0, src1, dst0, dst1, rows0, rows1,
              msg_sh, sidx, sgat,
              *, N, NP, C, EPT, RPT):
    c = lax.axis_index("c")
    s = lax.axis_index("s")
    NCH = EPT // C
    SD = 2 * C
    WPC = C * LANES

    def fill_zero(i, _):
        for k in range(HHID // LANES):
            rows0[i, pl.ds(k * LANES, LANES)] = jnp.zeros((LANES,),
                                                          jnp.float32)
        return 0
    lax.fori_loop(0, C, fill_zero, 0)
    zbase = s * RPT
    for t in range(RPT // C):
        pltpu.sync_copy(rows0, msg_sh.at[pl.ds(zbase + t * C, C)])
    if RPT % C:
        pltpu.sync_copy(rows0.at[pl.ds(0, RPT % C)],
                        msg_sh.at[pl.ds(zbase + (RPT // C) * C, RPT % C)])
    plsc.subcore_barrier()

    g0 = s * NCH
    gend = g0 + NCH
    coff = c * N

    def build(sd_v, src_v, dst_v):
        for j in range(C // LANES):
            src_v[pl.ds(j * LANES, LANES)] = (
                sd_v[pl.ds(j * LANES, LANES)] + coff)
            dst_v[pl.ds(j * LANES, LANES)] = sd_v[pl.ds(C + j * LANES, LANES)]

    def issue_idx(ci, sd_v, wp_v):
        pltpu.async_copy(sd_hbm.at[pl.ds(ci * SD, SD)], sd_v, sidx)
        pltpu.async_copy(wp_hbm.at[pl.ds(ci * WPC, WPC)], wp_v, sidx)

    def wait_idx(ci, sd_v, wp_v):
        pltpu.make_async_copy(sd_hbm.at[pl.ds(ci * SD, SD)], sd_v,
                              sidx).wait()
        pltpu.make_async_copy(wp_hbm.at[pl.ds(ci * WPC, WPC)], wp_v,
                              sidx).wait()

    def scale(rows_v, wp_v):
        def sc(e, _):
            wrow = wp_v[pl.ds(e * LANES, LANES)]
            for k in range(HHID // LANES):
                rows_v[e, pl.ds(k * LANES, LANES)] = (
                    rows_v[e, pl.ds(k * LANES, LANES)] * wrow)
            return 0
        lax.fori_loop(0, C, sc, 0, unroll=8)

    bufs = ((sd0, wp0, src0, dst0, rows0), (sd1, wp1, src1, dst1, rows1))

    pltpu.sync_copy(sd_hbm.at[pl.ds(g0 * SD, SD)], sd0)
    pltpu.sync_copy(wp_hbm.at[pl.ds(g0 * WPC, WPC)], wp0)
    build(sd0, src0, dst0)
    pltpu.async_copy(h_hbm.at[src0], rows0, sgat)
    issue_idx(g0 + 1, sd1, wp1)

    def sub(ci, A, B):
        sdA, wpA, srcA, dstA, rowsA = A
        sdB, wpB, srcB, dstB, rowsB = B
        pltpu.make_async_copy(h_hbm.at[srcA], rowsA, sgat).wait()
        scale(rowsA, wpA)
        wait_idx(ci + 1, sdB, wpB)
        build(sdB, srcB, dstB)
        pltpu.async_copy(h_hbm.at[srcB], rowsB, sgat)

        @pl.when(ci + 2 < gend)
        def _():
            issue_idx(ci + 2, sdA, wpA)
        pltpu.sync_copy(rowsA, msg_sh.at[dstA], add=True)

    def body(i, _):
        ci0 = g0 + i * 2
        sub(ci0, bufs[0], bufs[1])
        sub(ci0 + 1, bufs[1], bufs[0])
        return 0
    lax.fori_loop(0, (NCH - 1) // 2, body, 0)

    lb = (NCH - 1) % 2
    _, wpL, srcL, dstL, rowsL = bufs[lb]
    pltpu.make_async_copy(h_hbm.at[srcL], rowsL, sgat).wait()
    scale(rowsL, wpL)
    pltpu.sync_copy(rowsL, msg_sh.at[dstL], add=True)

    plsc.subcore_barrier()
    pltpu.sync_copy(msg_sh.at[pl.ds(s * RPT, RPT)],
                    msg_hbm.at[pl.ds(c * NP + s * RPT, RPT)])


def _msg(h2flat, sd, wp, N, NP):
    E = sd.shape[0] // 2
    C = CEDGE
    EPT = E // NS
    RPT = NP // NS
    assert (EPT // C) % 2 == 1 and EPT % C == 0
    body = functools.partial(_msg_body, N=N, NP=NP, C=C, EPT=EPT, RPT=RPT)
    f = pl.kernel(
        body,
        out_type=jax.ShapeDtypeStruct((NC * NP, HHID), jnp.float32),
        mesh=_mesh(),
        scratch_types=[
            pltpu.VMEM((2 * C,), jnp.int32),
            pltpu.VMEM((2 * C,), jnp.int32),
            pltpu.VMEM((C * LANES,), jnp.float32),
            pltpu.VMEM((C * LANES,), jnp.float32),
            pltpu.VMEM((C,), jnp.int32),
            pltpu.VMEM((C,), jnp.int32),
            pltpu.VMEM((C,), jnp.int32),
            pltpu.VMEM((C,), jnp.int32),
            pltpu.VMEM((C, HHID), jnp.float32),
            pltpu.VMEM((C, HHID), jnp.float32),
            pltpu.VMEM_SHARED((NP, HHID), jnp.float32),
            pltpu.SemaphoreType.DMA,
            pltpu.SemaphoreType.DMA,
        ],
    )
    return f(h2flat, sd, wp)


def _proj_split_body(x_ref, w_ref, b_ref, o_ref):
    y = jnp.dot(x_ref[...], w_ref[...],
                preferred_element_type=jnp.float32) + b_ref[...]
    o_ref[0] = y[:, :HHID]
    o_ref[1] = y[:, HHID:]


def _proj_split(x, W, b, BR=512):
    N, D = x.shape
    G = (N + BR - 1) // BR
    return pl.pallas_call(
        _proj_split_body,
        grid=(G,),
        in_specs=[
            pl.BlockSpec((BR, D), lambda i: (i, 0)),
            pl.BlockSpec((D, HID), lambda i: (0, 0)),
            pl.BlockSpec((1, HID), lambda i: (0, 0)),
        ],
        out_specs=pl.BlockSpec((NC, BR, HHID), lambda i: (0, i, 0)),
        out_shape=jax.ShapeDtypeStruct((NC, N, HHID), jnp.float32),
    )(x, W, b.reshape(1, HID))


def _gru_body(h_ref, m_ref, deg_ref, wz_ref, wr_ref, wh_ref, bz_ref, br_ref,
              bh_ref, g_ref, bt_ref, o_ref):
    h = jnp.concatenate([h_ref[0], h_ref[1]], axis=-1)
    m = jnp.concatenate([m_ref[0], m_ref[1]], axis=-1)
    m = m / jnp.maximum(deg_ref[...][:, :1], 1.0)
    hm = jnp.concatenate([h, m], axis=-1)
    z = jax.nn.sigmoid(jnp.dot(hm, wz_ref[...],
                               preferred_element_type=jnp.float32)
                       + bz_ref[...])
    r = jax.nn.sigmoid(jnp.dot(hm, wr_ref[...],
                               preferred_element_type=jnp.float32)
                       + br_ref[...])
    hr = jnp.concatenate([r * h, m], axis=-1)
    ht = jnp.tanh(jnp.dot(hr, wh_ref[...],
                          preferred_element_type=jnp.float32) + bh_ref[...])
    hn = (1.0 - z) * h + z * ht
    mu = jnp.mean(hn, axis=-1, keepdims=True)
    var = jnp.mean((hn - mu) ** 2, axis=-1, keepdims=True)
    y = (hn - mu) / jnp.sqrt(var + EPS_LN) * g_ref[...] + bt_ref[...]
    o_ref[0] = y[:, :HHID]
    o_ref[1] = y[:, HHID:]


def _gru(h2, msg2, deg, lp, BR=512):
    N = h2.shape[1]
    G = (N + BR - 1) // BR
    spec_w = pl.BlockSpec((2 * HID, HID), lambda i: (0, 0))
    spec_b = pl.BlockSpec((1, HID), lambda i: (0, 0))
    return pl.pallas_call(
        _gru_body,
        grid=(G,),
        in_specs=[
            pl.BlockSpec((NC, BR, HHID), lambda i: (0, i, 0)),
            pl.BlockSpec((NC, BR, HHID), lambda i: (0, i, 0)),
            pl.BlockSpec((BR, HHID), lambda i: (i, 0)),
            spec_w, spec_w, spec_w, spec_b, spec_b, spec_b, spec_b, spec_b,
        ],
        out_specs=pl.BlockSpec((NC, BR, HHID), lambda i: (0, i, 0)),
        out_shape=jax.ShapeDtypeStruct((NC, N, HHID), jnp.float32),
    )(h2, msg2, deg, lp['Wz'], lp['Wr'], lp['Wh'],
      lp['bz'].reshape(1, HID), lp['br'].reshape(1, HID),
      lp['bh'].reshape(1, HID), lp['g'].reshape(1, HID),
      lp['bt'].reshape(1, HID))


def _proj_out_body(h_ref, w_ref, b_ref, o_ref):
    h = jnp.concatenate([h_ref[0], h_ref[1]], axis=-1)
    o_ref[...] = jnp.dot(h, w_ref[...],
                         preferred_element_type=jnp.float32) + b_ref[...]


def _proj_out(h2, W, b, BR=512):
    N = h2.shape[1]
    G = (N + BR - 1) // BR
    return pl.pallas_call(
        _proj_out_body,
        grid=(G,),
        in_specs=[
            pl.BlockSpec((NC, BR, HHID), lambda i: (0, i, 0)),
            pl.BlockSpec((HID, HID), lambda i: (0, 0)),
            pl.BlockSpec((1, HID), lambda i: (0, 0)),
        ],
        out_specs=pl.BlockSpec((BR, HID), lambda i: (i, 0)),
        out_shape=jax.ShapeDtypeStruct((N, HID), jnp.float32),
    )(h2, W, b.reshape(1, HID))


def kernel(x, edge_index, path_lengths, merge_counts, params):
    N, D = x.shape
    E = edge_index.shape[1]
    src = edge_index[0].astype(jnp.int32)
    dst = edge_index[1].astype(jnp.int32)

    scale16 = jnp.broadcast_to(-1.0 / (params['tau'] + EPS_TAU),
                               (LANES,)).astype(jnp.float32)
    pw = jnp.concatenate([path_lengths.reshape(-1, CEDGE),
                          merge_counts.reshape(-1, CEDGE)],
                         axis=1).reshape(-1)
    deg_flat, wp, NP = _degwp(dst, pw, scale16, N)
    deg = deg_flat[:NP]
    sd = jnp.concatenate([src.reshape(-1, CEDGE), dst.reshape(-1, CEDGE)],
                         axis=1).reshape(-1)

    h2 = _proj_split(x, params['W_in'], params['b_in'])
    for lp in params['layers']:
        msg_flat = _msg(h2.reshape(NC * N, HHID), sd, wp, N, NP)
        msg2 = msg_flat.reshape(NC, NP, HHID)
        h2 = _gru(h2, msg2, deg, lp)
    return _proj_out(h2, params['W_out'], params['b_out'])


# speedup vs baseline: 1.7532x; 1.7532x over previous
import functools
import numpy as np
import jax
import jax.numpy as jnp
from jax import lax
from jax.experimental import pallas as pl
from jax.experimental.pallas import tpu as pltpu
from jax.experimental.pallas import tpu_sc as plsc

HID = 256
NL = 3
DECAY = 0.9
EPS_TAU = 1e-08
EPS_LN = 1e-05
LN_DECAY = float(np.log(DECAY))

NC = 2
NS = 16
LANES = 16
HHID = HID // 2
CEDGE = 80


def _mesh():
    return plsc.VectorSubcoreMesh(core_axis_name="c", subcore_axis_name="s")


def _degwp_body(dst_hbm, pw_hbm, scale_hbm, deg_hbm, wp_hbm,
                dst0, dst1, pw0, pw1, wq16_v, ones_v, scale_v, deg_sh, sidx,
                *, E, NP, C, EPT, RPT):
    c = lax.axis_index("c")
    s = lax.axis_index("s")
    NCH = EPT // C
    PW = 2 * C

    def fill(val, i, _):
        for k in range(HHID // LANES):
            ones_v[i, pl.ds(k * LANES, LANES)] = jnp.full((LANES,), val,
                                                          jnp.float32)
        return 0
    lax.fori_loop(0, C, functools.partial(fill, 0.0), 0)
    zbase = s * RPT
    for t in range(RPT // C):
        pltpu.sync_copy(ones_v, deg_sh.at[pl.ds(zbase + t * C, C)])
    if RPT % C:
        pltpu.sync_copy(ones_v.at[pl.ds(0, RPT % C)],
                        deg_sh.at[pl.ds(zbase + (RPT // C) * C, RPT % C)])
    lax.fori_loop(0, C, functools.partial(fill, 1.0), 0)
    pltpu.sync_copy(scale_hbm, scale_v)
    plsc.subcore_barrier()

    g0 = s * NCH
    gend = g0 + NCH

    def issue_idx(ci, dst_v, pw_v):
        pltpu.async_copy(dst_hbm.at[pl.ds(ci * C, C)], dst_v, sidx)
        pltpu.async_copy(pw_hbm.at[pl.ds(ci * PW, PW)], pw_v, sidx)

    def wait_idx(ci, dst_v, pw_v):
        pltpu.make_async_copy(dst_hbm.at[pl.ds(ci * C, C)], dst_v,
                              sidx).wait()
        pltpu.make_async_copy(pw_hbm.at[pl.ds(ci * PW, PW)], pw_v,
                              sidx).wait()

    def wq(ci, pw_v):
        @pl.when(c == 0)
        def _():
            for j in range(C // LANES):
                pv = pw_v[pl.ds(j * LANES, LANES)]
                mv = pw_v[pl.ds(C + j * LANES, LANES)]
                dist = pv * jnp.exp(mv * LN_DECAY)
                w = jnp.exp(dist * scale_v[...])
                for kk in range(LANES):
                    e = j * LANES + kk
                    wq16_v[pl.ds(e * LANES, LANES)] = (
                        jnp.broadcast_to(w[kk], (LANES,)))
            pltpu.sync_copy(wq16_v,
                            wp_hbm.at[pl.ds(ci * C * LANES, C * LANES)])

    bufs = ((dst0, pw0), (dst1, pw1))
    pltpu.sync_copy(dst_hbm.at[pl.ds(g0 * C, C)], dst0)
    pltpu.sync_copy(pw_hbm.at[pl.ds(g0 * PW, PW)], pw0)
    issue_idx(g0 + 1, dst1, pw1)

    def sub(ci, A, B):
        dstA, pwA = A
        dstB, pwB = B
        wq(ci, pwA)
        wait_idx(ci + 1, dstB, pwB)

        @pl.when(ci + 2 < gend)
        def _():
            issue_idx(ci + 2, dstA, pwA)
        pltpu.sync_copy(ones_v, deg_sh.at[dstA], add=True)

    def body(i, _):
        ci0 = g0 + i * 2
        sub(ci0, bufs[0], bufs[1])
        sub(ci0 + 1, bufs[1], bufs[0])
        return 0
    lax.fori_loop(0, (NCH - 1) // 2, body, 0)

    lb = (NCH - 1) % 2
    dstL, pwL = bufs[lb]
    wq(gend - 1, pwL)
    pltpu.sync_copy(ones_v, deg_sh.at[dstL], add=True)

    plsc.subcore_barrier()
    pltpu.sync_copy(deg_sh.at[pl.ds(s * RPT, RPT)],
                    deg_hbm.at[pl.ds(c * NP + s * RPT, RPT)])


def _degwp(dst, pw, scale16, N):
    E = dst.shape[0]
    NP = ((N + NS * 8 - 1) // (NS * 8)) * (NS * 8)
    C = CEDGE
    EPT = E // NS
    RPT = NP // NS
    assert (EPT // C) % 2 == 1 and EPT % C == 0
    body = functools.partial(_degwp_body, E=E, NP=NP, C=C, EPT=EPT, RPT=RPT)
    f = pl.kernel(
        body,
        out_type=(jax.ShapeDtypeStruct((NC * NP, HHID), jnp.float32),
                  jax.ShapeDtypeStruct((E * LANES,), jnp.float32)),
        mesh=_mesh(),
        scratch_types=[
            pltpu.VMEM((C,), jnp.int32),
            pltpu.VMEM((C,), jnp.int32),
            pltpu.VMEM((2 * C,), jnp.float32),
            pltpu.VMEM((2 * C,), jnp.float32),
            pltpu.VMEM((C * LANES,), jnp.float32),
            pltpu.VMEM((C, HHID), jnp.float32),
            pltpu.VMEM((LANES,), jnp.float32),
            pltpu.VMEM_SHARED((NP, HHID), jnp.float32),
            pltpu.SemaphoreType.DMA,
        ],
    )
    return f(dst, pw, scale16) + (NP,)


def _msg_body(h_hbm, sd_hbm, wp_hbm, msg_hbm,
              sd0, sd1, wp0, wp1, src0, src1, dst0, dst1, rows0, rows1,
              msg_sh, sidx, sgat,
              *, N, NP, C, EPT, RPT):
    c = lax.axis_index("c")
    s = lax.axis_index("s")
    NCH = EPT // C
    SD = 2 * C
    WPC = C * LANES

    def fill_zero(i, _):
        for k in range(HHID // LANES):
            rows0[i, pl.ds(k * LANES, LANES)] = jnp.zeros((LANES,),
                                                          jnp.float32)
        return 0
    lax.fori_loop(0, C, fill_zero, 0)
    zbase = s * RPT
    for t in range(RPT // C):
        pltpu.sync_copy(rows0, msg_sh.at[pl.ds(zbase + t * C, C)])
    if RPT % C:
        pltpu.sync_copy(rows0.at[pl.ds(0, RPT % C)],
                        msg_sh.at[pl.ds(zbase + (RPT // C) * C, RPT % C)])
    plsc.subcore_barrier()

    g0 = s * NCH
    gend = g0 + NCH
    coff = c * N

    def build(sd_v, src_v, dst_v):
        for j in range(C // LANES):
            src_v[pl.ds(j * LANES, LANES)] = (
                sd_v[pl.ds(j * LANES, LANES)] + coff)
            dst_v[pl.ds(j * LANES, LANES)] = sd_v[pl.ds(C + j * LANES, LANES)]

    def issue_idx(ci, sd_v, wp_v):
        pltpu.async_copy(sd_hbm.at[pl.ds(ci * SD, SD)], sd_v, sidx)
        pltpu.async_copy(wp_hbm.at[pl.ds(ci * WPC, WPC)], wp_v, sidx)

    def wait_idx(ci, sd_v, wp_v):
        pltpu.make_async_copy(sd_hbm.at[pl.ds(ci * SD, SD)], sd_v,
                              sidx).wait()
        pltpu.make_async_copy(wp_hbm.at[pl.ds(ci * WPC, WPC)], wp_v,
                              sidx).wait()

    def scale(rows_v, wp_v):
        def sc(e, _):
            wrow = wp_v[pl.ds(e * LANES, LANES)]
            for k in range(HHID // LANES):
                rows_v[e, pl.ds(k * LANES, LANES)] = (
                    rows_v[e, pl.ds(k * LANES, LANES)] * wrow)
            return 0
        lax.fori_loop(0, C, sc, 0, unroll=2)

    bufs = ((sd0, wp0, src0, dst0, rows0), (sd1, wp1, src1, dst1, rows1))

    pltpu.sync_copy(sd_hbm.at[pl.ds(g0 * SD, SD)], sd0)
    pltpu.sync_copy(wp_hbm.at[pl.ds(g0 * WPC, WPC)], wp0)
    build(sd0, src0, dst0)
    pltpu.async_copy(h_hbm.at[src0], rows0, sgat)
    issue_idx(g0 + 1, sd1, wp1)

    def sub(ci, A, B):
        sdA, wpA, srcA, dstA, rowsA = A
        sdB, wpB, srcB, dstB, rowsB = B
        pltpu.make_async_copy(h_hbm.at[srcA], rowsA, sgat).wait()
        scale(rowsA, wpA)
        wait_idx(ci + 1, sdB, wpB)
        build(sdB, srcB, dstB)
        pltpu.async_copy(h_hbm.at[srcB], rowsB, sgat)

        @pl.when(ci + 2 < gend)
        def _():
            issue_idx(ci + 2, sdA, wpA)
        pltpu.sync_copy(rowsA, msg_sh.at[dstA], add=True)

    def body(i, _):
        ci0 = g0 + i * 2
        sub(ci0, bufs[0], bufs[1])
        sub(ci0 + 1, bufs[1], bufs[0])
        return 0
    lax.fori_loop(0, (NCH - 1) // 2, body, 0)

    lb = (NCH - 1) % 2
    _, wpL, srcL, dstL, rowsL = bufs[lb]
    pltpu.make_async_copy(h_hbm.at[srcL], rowsL, sgat).wait()
    scale(rowsL, wpL)
    pltpu.sync_copy(rowsL, msg_sh.at[dstL], add=True)

    plsc.subcore_barrier()
    pltpu.sync_copy(msg_sh.at[pl.ds(s * RPT, RPT)],
                    msg_hbm.at[pl.ds(c * NP + s * RPT, RPT)])


def _msg(h2flat, sd, wp, N, NP):
    E = sd.shape[0] // 2
    C = CEDGE
    EPT = E // NS
    RPT = NP // NS
    assert (EPT // C) % 2 == 1 and EPT % C == 0
    body = functools.partial(_msg_body, N=N, NP=NP, C=C, EPT=EPT, RPT=RPT)
    f = pl.kernel(
        body,
        out_type=jax.ShapeDtypeStruct((NC * NP, HHID), jnp.float32),
        mesh=_mesh(),
        scratch_types=[
            pltpu.VMEM((2 * C,), jnp.int32),
            pltpu.VMEM((2 * C,), jnp.int32),
            pltpu.VMEM((C * LANES,), jnp.float32),
            pltpu.VMEM((C * LANES,), jnp.float32),
            pltpu.VMEM((C,), jnp.int32),
            pltpu.VMEM((C,), jnp.int32),
            pltpu.VMEM((C,), jnp.int32),
            pltpu.VMEM((C,), jnp.int32),
            pltpu.VMEM((C, HHID), jnp.float32),
            pltpu.VMEM((C, HHID), jnp.float32),
            pltpu.VMEM_SHARED((NP, HHID), jnp.float32),
            pltpu.SemaphoreType.DMA,
            pltpu.SemaphoreType.DMA,
        ],
    )
    return f(h2flat, sd, wp)


def _proj_split_body(x_ref, w_ref, b_ref, o_ref):
    y = jnp.dot(x_ref[...], w_ref[...],
                preferred_element_type=jnp.float32) + b_ref[...]
    o_ref[0] = y[:, :HHID]
    o_ref[1] = y[:, HHID:]


def _proj_split(x, W, b, BR=512):
    N, D = x.shape
    G = (N + BR - 1) // BR
    return pl.pallas_call(
        _proj_split_body,
        grid=(G,),
        in_specs=[
            pl.BlockSpec((BR, D), lambda i: (i, 0)),
            pl.BlockSpec((D, HID), lambda i: (0, 0)),
            pl.BlockSpec((1, HID), lambda i: (0, 0)),
        ],
        out_specs=pl.BlockSpec((NC, BR, HHID), lambda i: (0, i, 0)),
        out_shape=jax.ShapeDtypeStruct((NC, N, HHID), jnp.float32),
    )(x, W, b.reshape(1, HID))


def _gru_body(h_ref, m_ref, deg_ref, wz_ref, wr_ref, wh_ref, bz_ref, br_ref,
              bh_ref, g_ref, bt_ref, o_ref):
    h = jnp.concatenate([h_ref[0], h_ref[1]], axis=-1)
    m = jnp.concatenate([m_ref[0], m_ref[1]], axis=-1)
    m = m / jnp.maximum(deg_ref[...][:, :1], 1.0)
    hm = jnp.concatenate([h, m], axis=-1)
    z = jax.nn.sigmoid(jnp.dot(hm, wz_ref[...],
                               preferred_element_type=jnp.float32)
                       + bz_ref[...])
    r = jax.nn.sigmoid(jnp.dot(hm, wr_ref[...],
                               preferred_element_type=jnp.float32)
                       + br_ref[...])
    hr = jnp.concatenate([r * h, m], axis=-1)
    ht = jnp.tanh(jnp.dot(hr, wh_ref[...],
                          preferred_element_type=jnp.float32) + bh_ref[...])
    hn = (1.0 - z) * h + z * ht
    mu = jnp.mean(hn, axis=-1, keepdims=True)
    var = jnp.mean((hn - mu) ** 2, axis=-1, keepdims=True)
    y = (hn - mu) / jnp.sqrt(var + EPS_LN) * g_ref[...] + bt_ref[...]
    o_ref[0] = y[:, :HHID]
    o_ref[1] = y[:, HHID:]


def _gru(h2, msg2, deg, lp, BR=512):
    N = h2.shape[1]
    G = (N + BR - 1) // BR
    spec_w = pl.BlockSpec((2 * HID, HID), lambda i: (0, 0))
    spec_b = pl.BlockSpec((1, HID), lambda i: (0, 0))
    return pl.pallas_call(
        _gru_body,
        grid=(G,),
        in_specs=[
            pl.BlockSpec((NC, BR, HHID), lambda i: (0, i, 0)),
            pl.BlockSpec((NC, BR, HHID), lambda i: (0, i, 0)),
            pl.BlockSpec((BR, HHID), lambda i: (i, 0)),
            spec_w, spec_w, spec_w, spec_b, spec_b, spec_b, spec_b, spec_b,
        ],
        out_specs=pl.BlockSpec((NC, BR, HHID), lambda i: (0, i, 0)),
        out_shape=jax.ShapeDtypeStruct((NC, N, HHID), jnp.float32),
    )(h2, msg2, deg, lp['Wz'], lp['Wr'], lp['Wh'],
      lp['bz'].reshape(1, HID), lp['br'].reshape(1, HID),
      lp['bh'].reshape(1, HID), lp['g'].reshape(1, HID),
      lp['bt'].reshape(1, HID))


def _proj_out_body(h_ref, w_ref, b_ref, o_ref):
    h = jnp.concatenate([h_ref[0], h_ref[1]], axis=-1)
    o_ref[...] = jnp.dot(h, w_ref[...],
                         preferred_element_type=jnp.float32) + b_ref[...]


def _proj_out(h2, W, b, BR=512):
    N = h2.shape[1]
    G = (N + BR - 1) // BR
    return pl.pallas_call(
        _proj_out_body,
        grid=(G,),
        in_specs=[
            pl.BlockSpec((NC, BR, HHID), lambda i: (0, i, 0)),
            pl.BlockSpec((HID, HID), lambda i: (0, 0)),
            pl.BlockSpec((1, HID), lambda i: (0, 0)),
        ],
        out_specs=pl.BlockSpec((BR, HID), lambda i: (i, 0)),
        out_shape=jax.ShapeDtypeStruct((N, HID), jnp.float32),
    )(h2, W, b.reshape(1, HID))


def kernel(x, edge_index, path_lengths, merge_counts, params):
    N, D = x.shape
    E = edge_index.shape[1]
    src = edge_index[0].astype(jnp.int32)
    dst = edge_index[1].astype(jnp.int32)

    scale16 = jnp.broadcast_to(-1.0 / (params['tau'] + EPS_TAU),
                               (LANES,)).astype(jnp.float32)
    pw = jnp.concatenate([path_lengths.reshape(-1, CEDGE),
                          merge_counts.reshape(-1, CEDGE)],
                         axis=1).reshape(-1)
    deg_flat, wp, NP = _degwp(dst, pw, scale16, N)
    deg = deg_flat[:NP]
    sd = jnp.concatenate([src.reshape(-1, CEDGE), dst.reshape(-1, CEDGE)],
                         axis=1).reshape(-1)

    h2 = _proj_split(x, params['W_in'], params['b_in'])
    for lp in params['layers']:
        msg_flat = _msg(h2.reshape(NC * N, HHID), sd, wp, N, NP)
        msg2 = msg_flat.reshape(NC, NP, HHID)
        h2 = _gru(h2, msg2, deg, lp)
    return _proj_out(h2, params['W_out'], params['b_out'])


# speedup vs baseline: 2.1486x; 1.2255x over previous
import functools
import numpy as np
import jax
import jax.numpy as jnp
from jax import lax
from jax.experimental import pallas as pl
from jax.experimental.pallas import tpu as pltpu
from jax.experimental.pallas import tpu_sc as plsc

HID = 256
NL = 3
DECAY = 0.9
EPS_TAU = 1e-08
EPS_LN = 1e-05
LN_DECAY = float(np.log(DECAY))

NC = 2
NS = 16
LANES = 16
HHID = HID // 2
CEDGE = 80


def _mesh():
    return plsc.VectorSubcoreMesh(core_axis_name="c", subcore_axis_name="s")


def _degwp_body(dst_hbm, pw_hbm, scale_hbm, deg_hbm, wp_hbm,
                dst0, dst1, pw0, pw1, wq16_v, ones_v, scale_v, deg_sh, sidx,
                *, E, NP, C, EPT, RPT):
    c = lax.axis_index("c")
    s = lax.axis_index("s")
    NCH = EPT // C
    PW = 2 * C

    def fill(val, i, _):
        for k in range(HHID // LANES):
            ones_v[i, pl.ds(k * LANES, LANES)] = jnp.full((LANES,), val,
                                                          jnp.float32)
        return 0
    lax.fori_loop(0, C, functools.partial(fill, 0.0), 0)
    zbase = s * RPT
    for t in range(RPT // C):
        pltpu.sync_copy(ones_v, deg_sh.at[pl.ds(zbase + t * C, C)])
    if RPT % C:
        pltpu.sync_copy(ones_v.at[pl.ds(0, RPT % C)],
                        deg_sh.at[pl.ds(zbase + (RPT // C) * C, RPT % C)])
    lax.fori_loop(0, C, functools.partial(fill, 1.0), 0)
    pltpu.sync_copy(scale_hbm, scale_v)
    plsc.subcore_barrier()

    g0 = s * NCH
    gend = g0 + NCH

    def issue_idx(ci, dst_v, pw_v):
        pltpu.async_copy(dst_hbm.at[pl.ds(ci * C, C)], dst_v, sidx)
        pltpu.async_copy(pw_hbm.at[pl.ds(ci * PW, PW)], pw_v, sidx)

    def wait_idx(ci, dst_v, pw_v):
        pltpu.make_async_copy(dst_hbm.at[pl.ds(ci * C, C)], dst_v,
                              sidx).wait()
        pltpu.make_async_copy(pw_hbm.at[pl.ds(ci * PW, PW)], pw_v,
                              sidx).wait()

    def wq(ci, pw_v):
        @pl.when(c == 0)
        def _():
            for j in range(C // LANES):
                pv = pw_v[pl.ds(j * LANES, LANES)]
                mv = pw_v[pl.ds(C + j * LANES, LANES)]
                dist = pv * jnp.exp(mv * LN_DECAY)
                w = jnp.exp(dist * scale_v[...])
                for kk in range(LANES):
                    e = j * LANES + kk
                    wq16_v[pl.ds(e * LANES, LANES)] = (
                        jnp.broadcast_to(w[kk], (LANES,)))
            pltpu.sync_copy(wq16_v,
                            wp_hbm.at[pl.ds(ci * C * LANES, C * LANES)])

    bufs = ((dst0, pw0), (dst1, pw1))
    pltpu.sync_copy(dst_hbm.at[pl.ds(g0 * C, C)], dst0)
    pltpu.sync_copy(pw_hbm.at[pl.ds(g0 * PW, PW)], pw0)
    issue_idx(g0 + 1, dst1, pw1)

    def sub(ci, A, B):
        dstA, pwA = A
        dstB, pwB = B
        wq(ci, pwA)
        wait_idx(ci + 1, dstB, pwB)

        @pl.when(ci + 2 < gend)
        def _():
            issue_idx(ci + 2, dstA, pwA)
        pltpu.sync_copy(ones_v, deg_sh.at[dstA], add=True)

    def body(i, _):
        ci0 = g0 + i * 2
        sub(ci0, bufs[0], bufs[1])
        sub(ci0 + 1, bufs[1], bufs[0])
        return 0
    lax.fori_loop(0, (NCH - 1) // 2, body, 0)

    lb = (NCH - 1) % 2
    dstL, pwL = bufs[lb]
    wq(gend - 1, pwL)
    pltpu.sync_copy(ones_v, deg_sh.at[dstL], add=True)

    plsc.subcore_barrier()
    pltpu.sync_copy(deg_sh.at[pl.ds(s * RPT, RPT)],
                    deg_hbm.at[pl.ds(c * NP + s * RPT, RPT)])


def _degwp(dst, pw, scale16, N):
    E = dst.shape[0]
    NP = ((N + NS * 8 - 1) // (NS * 8)) * (NS * 8)
    C = CEDGE
    EPT = E // NS
    RPT = NP // NS
    assert (EPT // C) % 2 == 1 and EPT % C == 0
    body = functools.partial(_degwp_body, E=E, NP=NP, C=C, EPT=EPT, RPT=RPT)
    f = pl.kernel(
        body,
        out_type=(jax.ShapeDtypeStruct((NC * NP, HHID), jnp.float32),
                  jax.ShapeDtypeStruct((E * LANES,), jnp.float32)),
        mesh=_mesh(),
        scratch_types=[
            pltpu.VMEM((C,), jnp.int32),
            pltpu.VMEM((C,), jnp.int32),
            pltpu.VMEM((2 * C,), jnp.float32),
            pltpu.VMEM((2 * C,), jnp.float32),
            pltpu.VMEM((C * LANES,), jnp.float32),
            pltpu.VMEM((C, HHID), jnp.float32),
            pltpu.VMEM((LANES,), jnp.float32),
            pltpu.VMEM_SHARED((NP, HHID), jnp.float32),
            pltpu.SemaphoreType.DMA,
        ],
    )
    return f(dst, pw, scale16) + (NP,)


def _msg_body(h_hbm, sd_hbm, wp_hbm, msg_hbm,
              sd0, sd1, wp0, wp1, src0, src1, dst0, dst1, rows0, rows1,
              msg_sh, sidx, sgat, ssc,
              *, N, NP, C, EPT, RPT):
    c = lax.axis_index("c")
    s = lax.axis_index("s")
    NCH = EPT // C
    SD = 2 * C
    WPC = C * LANES

    def fill_zero(i, _):
        for k in range(HHID // LANES):
            rows0[i, pl.ds(k * LANES, LANES)] = jnp.zeros((LANES,),
                                                          jnp.float32)
        return 0
    lax.fori_loop(0, C, fill_zero, 0)
    zbase = s * RPT
    for t in range(RPT // C):
        pltpu.sync_copy(rows0, msg_sh.at[pl.ds(zbase + t * C, C)])
    if RPT % C:
        pltpu.sync_copy(rows0.at[pl.ds(0, RPT % C)],
                        msg_sh.at[pl.ds(zbase + (RPT // C) * C, RPT % C)])
    plsc.subcore_barrier()

    g0 = s * NCH
    gend = g0 + NCH
    coff = c * N

    def build(sd_v, src_v, dst_v):
        for j in range(C // LANES):
            src_v[pl.ds(j * LANES, LANES)] = (
                sd_v[pl.ds(j * LANES, LANES)] + coff)
            dst_v[pl.ds(j * LANES, LANES)] = sd_v[pl.ds(C + j * LANES, LANES)]

    def issue_idx(ci, sd_v, wp_v):
        pltpu.async_copy(sd_hbm.at[pl.ds(ci * SD, SD)], sd_v, sidx)
        pltpu.async_copy(wp_hbm.at[pl.ds(ci * WPC, WPC)], wp_v, sidx)

    def wait_idx(ci, sd_v, wp_v):
        pltpu.make_async_copy(sd_hbm.at[pl.ds(ci * SD, SD)], sd_v,
                              sidx).wait()
        pltpu.make_async_copy(wp_hbm.at[pl.ds(ci * WPC, WPC)], wp_v,
                              sidx).wait()

    def scale(rows_v, wp_v):
        def sc(e, _):
            wrow = wp_v[pl.ds(e * LANES, LANES)]
            for k in range(HHID // LANES):
                rows_v[e, pl.ds(k * LANES, LANES)] = (
                    rows_v[e, pl.ds(k * LANES, LANES)] * wrow)
            return 0
        lax.fori_loop(0, C, sc, 0, unroll=2)

    bufs = ((sd0, wp0, src0, dst0, rows0), (sd1, wp1, src1, dst1, rows1))

    pltpu.sync_copy(sd_hbm.at[pl.ds(g0 * SD, SD)], sd0)
    pltpu.sync_copy(wp_hbm.at[pl.ds(g0 * WPC, WPC)], wp0)
    build(sd0, src0, dst0)
    pltpu.async_copy(h_hbm.at[src0], rows0, sgat)
    issue_idx(g0 + 1, sd1, wp1)

    def sub(ci, A, B):
        sdA, wpA, srcA, dstA, rowsA = A
        sdB, wpB, srcB, dstB, rowsB = B
        pltpu.make_async_copy(h_hbm.at[srcA], rowsA, sgat).wait()

        @pl.when(ci > g0)
        def _():
            pltpu.make_async_copy(rowsB, msg_sh.at[dstB], ssc).wait()
        wait_idx(ci + 1, sdB, wpB)
        build(sdB, srcB, dstB)
        pltpu.async_copy(h_hbm.at[srcB], rowsB, sgat)
        scale(rowsA, wpA)

        @pl.when(ci + 2 < gend)
        def _():
            issue_idx(ci + 2, sdA, wpA)
        pltpu.async_copy(rowsA, msg_sh.at[dstA], ssc, add=True)

    def body(i, _):
        ci0 = g0 + i * 2
        sub(ci0, bufs[0], bufs[1])
        sub(ci0 + 1, bufs[1], bufs[0])
        return 0
    lax.fori_loop(0, (NCH - 1) // 2, body, 0)

    lb = (NCH - 1) % 2
    _, wpL, srcL, dstL, rowsL = bufs[lb]
    sdP, wpP, srcP, dstP, rowsP = bufs[1 - lb]
    pltpu.make_async_copy(h_hbm.at[srcL], rowsL, sgat).wait()
    pltpu.make_async_copy(rowsP, msg_sh.at[dstP], ssc).wait()
    scale(rowsL, wpL)
    pltpu.sync_copy(rowsL, msg_sh.at[dstL], add=True)

    plsc.subcore_barrier()
    pltpu.sync_copy(msg_sh.at[pl.ds(s * RPT, RPT)],
                    msg_hbm.at[pl.ds(c * NP + s * RPT, RPT)])


def _msg(h2flat, sd, wp, N, NP):
    E = sd.shape[0] // 2
    C = CEDGE
    EPT = E // NS
    RPT = NP // NS
    assert (EPT // C) % 2 == 1 and EPT % C == 0
    body = functools.partial(_msg_body, N=N, NP=NP, C=C, EPT=EPT, RPT=RPT)
    f = pl.kernel(
        body,
        out_type=jax.ShapeDtypeStruct((NC * NP, HHID), jnp.float32),
        mesh=_mesh(),
        scratch_types=[
            pltpu.VMEM((2 * C,), jnp.int32),
            pltpu.VMEM((2 * C,), jnp.int32),
            pltpu.VMEM((C * LANES,), jnp.float32),
            pltpu.VMEM((C * LANES,), jnp.float32),
            pltpu.VMEM((C,), jnp.int32),
            pltpu.VMEM((C,), jnp.int32),
            pltpu.VMEM((C,), jnp.int32),
            pltpu.VMEM((C,), jnp.int32),
            pltpu.VMEM((C, HHID), jnp.float32),
            pltpu.VMEM((C, HHID), jnp.float32),
            pltpu.VMEM_SHARED((NP, HHID), jnp.float32),
            pltpu.SemaphoreType.DMA,
            pltpu.SemaphoreType.DMA,
            pltpu.SemaphoreType.DMA,
        ],
    )
    return f(h2flat, sd, wp)


def _proj_split_body(x_ref, w_ref, b_ref, o_ref):
    y = jnp.dot(x_ref[...], w_ref[...],
                preferred_element_type=jnp.float32) + b_ref[...]
    o_ref[0] = y[:, :HHID]
    o_ref[1] = y[:, HHID:]


def _proj_split(x, W, b, BR=512):
    N, D = x.shape
    G = (N + BR - 1) // BR
    return pl.pallas_call(
        _proj_split_body,
        grid=(G,),
        in_specs=[
            pl.BlockSpec((BR, D), lambda i: (i, 0)),
            pl.BlockSpec((D, HID), lambda i: (0, 0)),
            pl.BlockSpec((1, HID), lambda i: (0, 0)),
        ],
        out_specs=pl.BlockSpec((NC, BR, HHID), lambda i: (0, i, 0)),
        out_shape=jax.ShapeDtypeStruct((NC, N, HHID), jnp.float32),
    )(x, W, b.reshape(1, HID))


def _gru_body(h_ref, m_ref, deg_ref, wz_ref, wr_ref, wh_ref, bz_ref, br_ref,
              bh_ref, g_ref, bt_ref, o_ref):
    h = jnp.concatenate([h_ref[0], h_ref[1]], axis=-1)
    m = jnp.concatenate([m_ref[0], m_ref[1]], axis=-1)
    m = m / jnp.maximum(deg_ref[...][:, :1], 1.0)
    hm = jnp.concatenate([h, m], axis=-1)
    z = jax.nn.sigmoid(jnp.dot(hm, wz_ref[...],
                               preferred_element_type=jnp.float32)
                       + bz_ref[...])
    r = jax.nn.sigmoid(jnp.dot(hm, wr_ref[...],
                               preferred_element_type=jnp.float32)
                       + br_ref[...])
    hr = jnp.concatenate([r * h, m], axis=-1)
    ht = jnp.tanh(jnp.dot(hr, wh_ref[...],
                          preferred_element_type=jnp.float32) + bh_ref[...])
    hn = (1.0 - z) * h + z * ht
    mu = jnp.mean(hn, axis=-1, keepdims=True)
    var = jnp.mean((hn - mu) ** 2, axis=-1, keepdims=True)
    y = (hn - mu) / jnp.sqrt(var + EPS_LN) * g_ref[...] + bt_ref[...]
    o_ref[0] = y[:, :HHID]
    o_ref[1] = y[:, HHID:]


def _gru(h2, msg2, deg, lp, BR=512):
    N = h2.shape[1]
    G = (N + BR - 1) // BR
    spec_w = pl.BlockSpec((2 * HID, HID), lambda i: (0, 0))
    spec_b = pl.BlockSpec((1, HID), lambda i: (0, 0))
    return pl.pallas_call(
        _gru_body,
        grid=(G,),
        in_specs=[
            pl.BlockSpec((NC, BR, HHID), lambda i: (0, i, 0)),
            pl.BlockSpec((NC, BR, HHID), lambda i: (0, i, 0)),
            pl.BlockSpec((BR, HHID), lambda i: (i, 0)),
            spec_w, spec_w, spec_w, spec_b, spec_b, spec_b, spec_b, spec_b,
        ],
        out_specs=pl.BlockSpec((NC, BR, HHID), lambda i: (0, i, 0)),
        out_shape=jax.ShapeDtypeStruct((NC, N, HHID), jnp.float32),
    )(h2, msg2, deg, lp['Wz'], lp['Wr'], lp['Wh'],
      lp['bz'].reshape(1, HID), lp['br'].reshape(1, HID),
      lp['bh'].reshape(1, HID), lp['g'].reshape(1, HID),
      lp['bt'].reshape(1, HID))


def _proj_out_body(h_ref, w_ref, b_ref, o_ref):
    h = jnp.concatenate([h_ref[0], h_ref[1]], axis=-1)
    o_ref[...] = jnp.dot(h, w_ref[...],
                         preferred_element_type=jnp.float32) + b_ref[...]


def _proj_out(h2, W, b, BR=512):
    N = h2.shape[1]
    G = (N + BR - 1) // BR
    return pl.pallas_call(
        _proj_out_body,
        grid=(G,),
        in_specs=[
            pl.BlockSpec((NC, BR, HHID), lambda i: (0, i, 0)),
            pl.BlockSpec((HID, HID), lambda i: (0, 0)),
            pl.BlockSpec((1, HID), lambda i: (0, 0)),
        ],
        out_specs=pl.BlockSpec((BR, HID), lambda i: (i, 0)),
        out_shape=jax.ShapeDtypeStruct((N, HID), jnp.float32),
    )(h2, W, b.reshape(1, HID))


def kernel(x, edge_index, path_lengths, merge_counts, params):
    N, D = x.shape
    E = edge_index.shape[1]
    src = edge_index[0].astype(jnp.int32)
    dst = edge_index[1].astype(jnp.int32)

    scale16 = jnp.broadcast_to(-1.0 / (params['tau'] + EPS_TAU),
                               (LANES,)).astype(jnp.float32)
    pw = jnp.concatenate([path_lengths.reshape(-1, CEDGE),
                          merge_counts.reshape(-1, CEDGE)],
                         axis=1).reshape(-1)
    deg_flat, wp, NP = _degwp(dst, pw, scale16, N)
    deg = deg_flat[:NP]
    sd = jnp.concatenate([src.reshape(-1, CEDGE), dst.reshape(-1, CEDGE)],
                         axis=1).reshape(-1)

    h2 = _proj_split(x, params['W_in'], params['b_in'])
    for lp in params['layers']:
        msg_flat = _msg(h2.reshape(NC * N, HHID), sd, wp, N, NP)
        msg2 = msg_flat.reshape(NC, NP, HHID)
        h2 = _gru(h2, msg2, deg, lp)
    return _proj_out(h2, params['W_out'], params['b_out'])


# speedup vs baseline: 2.1873x; 1.0180x over previous
import functools
import numpy as np
import jax
import jax.numpy as jnp
from jax import lax
from jax.experimental import pallas as pl
from jax.experimental.pallas import tpu as pltpu
from jax.experimental.pallas import tpu_sc as plsc

HID = 256
NL = 3
DECAY = 0.9
EPS_TAU = 1e-08
EPS_LN = 1e-05
LN_DECAY = float(np.log(DECAY))

NC = 2
NS = 16
LANES = 16
HHID = HID // 2
CEDGE = 80


def _mesh():
    return plsc.VectorSubcoreMesh(core_axis_name="c", subcore_axis_name="s")


def _degwp_body(dst_hbm, pw_hbm, scale_hbm, deg_hbm, wp_hbm,
                dst0, dst1, pw0, pw1, wq16_v, ones_v, scale_v, deg_sh, sidx,
                *, E, NP, C, EPT, RPT):
    c = lax.axis_index("c")
    s = lax.axis_index("s")
    NCH = EPT // C
    PW = 2 * C

    def fill(val, i, _):
        for k in range(HHID // LANES):
            ones_v[i, pl.ds(k * LANES, LANES)] = jnp.full((LANES,), val,
                                                          jnp.float32)
        return 0
    lax.fori_loop(0, C, functools.partial(fill, 0.0), 0)
    zbase = s * RPT
    for t in range(RPT // C):
        pltpu.sync_copy(ones_v, deg_sh.at[pl.ds(zbase + t * C, C)])
    if RPT % C:
        pltpu.sync_copy(ones_v.at[pl.ds(0, RPT % C)],
                        deg_sh.at[pl.ds(zbase + (RPT // C) * C, RPT % C)])
    lax.fori_loop(0, C, functools.partial(fill, 1.0), 0)
    pltpu.sync_copy(scale_hbm, scale_v)
    plsc.subcore_barrier()

    g0 = s * NCH
    gend = g0 + NCH

    def issue_idx(ci, dst_v, pw_v):
        pltpu.async_copy(dst_hbm.at[pl.ds(ci * C, C)], dst_v, sidx)
        pltpu.async_copy(pw_hbm.at[pl.ds(ci * PW, PW)], pw_v, sidx)

    def wait_idx(ci, dst_v, pw_v):
        pltpu.make_async_copy(dst_hbm.at[pl.ds(ci * C, C)], dst_v,
                              sidx).wait()
        pltpu.make_async_copy(pw_hbm.at[pl.ds(ci * PW, PW)], pw_v,
                              sidx).wait()

    def wq(ci, pw_v):
        @pl.when(c == 0)
        def _():
            for j in range(C // LANES):
                pv = pw_v[pl.ds(j * LANES, LANES)]
                mv = pw_v[pl.ds(C + j * LANES, LANES)]
                dist = pv * jnp.exp(mv * LN_DECAY)
                w = jnp.exp(dist * scale_v[...])
                for kk in range(LANES):
                    e = j * LANES + kk
                    wq16_v[pl.ds(e * LANES, LANES)] = (
                        jnp.broadcast_to(w[kk], (LANES,)))
            pltpu.sync_copy(wq16_v,
                            wp_hbm.at[pl.ds(ci * C * LANES, C * LANES)])

    bufs = ((dst0, pw0), (dst1, pw1))
    pltpu.sync_copy(dst_hbm.at[pl.ds(g0 * C, C)], dst0)
    pltpu.sync_copy(pw_hbm.at[pl.ds(g0 * PW, PW)], pw0)
    issue_idx(g0 + 1, dst1, pw1)

    def sub(ci, A, B):
        dstA, pwA = A
        dstB, pwB = B
        wq(ci, pwA)
        wait_idx(ci + 1, dstB, pwB)

        @pl.when(ci + 2 < gend)
        def _():
            issue_idx(ci + 2, dstA, pwA)
        pltpu.sync_copy(ones_v, deg_sh.at[dstA], add=True)

    def body(i, _):
        ci0 = g0 + i * 2
        sub(ci0, bufs[0], bufs[1])
        sub(ci0 + 1, bufs[1], bufs[0])
        return 0
    lax.fori_loop(0, (NCH - 1) // 2, body, 0)

    lb = (NCH - 1) % 2
    dstL, pwL = bufs[lb]
    wq(gend - 1, pwL)
    pltpu.sync_copy(ones_v, deg_sh.at[dstL], add=True)

    plsc.subcore_barrier()
    pltpu.sync_copy(deg_sh.at[pl.ds(s * RPT, RPT)],
                    deg_hbm.at[pl.ds(c * NP + s * RPT, RPT)])


def _degwp(dst, pw, scale16, N):
    E = dst.shape[0]
    NP = ((N + NS * 8 - 1) // (NS * 8)) * (NS * 8)
    C = CEDGE
    EPT = E // NS
    RPT = NP // NS
    assert (EPT // C) % 2 == 1 and EPT % C == 0
    body = functools.partial(_degwp_body, E=E, NP=NP, C=C, EPT=EPT, RPT=RPT)
    f = pl.kernel(
        body,
        out_type=(jax.ShapeDtypeStruct((NC * NP, HHID), jnp.float32),
                  jax.ShapeDtypeStruct((E * LANES,), jnp.float32)),
        mesh=_mesh(),
        scratch_types=[
            pltpu.VMEM((C,), jnp.int32),
            pltpu.VMEM((C,), jnp.int32),
            pltpu.VMEM((2 * C,), jnp.float32),
            pltpu.VMEM((2 * C,), jnp.float32),
            pltpu.VMEM((C * LANES,), jnp.float32),
            pltpu.VMEM((C, HHID), jnp.float32),
            pltpu.VMEM((LANES,), jnp.float32),
            pltpu.VMEM_SHARED((NP, HHID), jnp.float32),
            pltpu.SemaphoreType.DMA,
        ],
    )
    return f(dst, pw, scale16) + (NP,)


def _msg_body(h_hbm, sd_hbm, wp_hbm, msg_hbm,
              sd0, sd1, wp0, wp1, src0, src1, dst0, dst1, rows0, rows1,
              msg_sh, sidx, sgat, ssc,
              *, N, NP, C, EPT, RPT):
    c = lax.axis_index("c")
    s = lax.axis_index("s")
    NCH = EPT // C
    SD = 2 * C
    WPC = C * LANES

    def fill_zero(i, _):
        for k in range(HHID // LANES):
            rows0[i, pl.ds(k * LANES, LANES)] = jnp.zeros((LANES,),
                                                          jnp.float32)
        return 0
    lax.fori_loop(0, C, fill_zero, 0)
    zbase = s * RPT
    for t in range(RPT // C):
        pltpu.sync_copy(rows0, msg_sh.at[pl.ds(zbase + t * C, C)])
    if RPT % C:
        pltpu.sync_copy(rows0.at[pl.ds(0, RPT % C)],
                        msg_sh.at[pl.ds(zbase + (RPT // C) * C, RPT % C)])
    plsc.subcore_barrier()

    g0 = s * NCH
    gend = g0 + NCH
    coff = c * N

    def build(sd_v, src_v, dst_v):
        for j in range(C // LANES):
            src_v[pl.ds(j * LANES, LANES)] = (
                sd_v[pl.ds(j * LANES, LANES)] + coff)
            dst_v[pl.ds(j * LANES, LANES)] = sd_v[pl.ds(C + j * LANES, LANES)]

    def issue_idx(ci, sd_v, wp_v):
        pltpu.async_copy(sd_hbm.at[pl.ds(ci * SD, SD)], sd_v, sidx)
        pltpu.async_copy(wp_hbm.at[pl.ds(ci * WPC, WPC)], wp_v, sidx)

    def wait_idx(ci, sd_v, wp_v):
        pltpu.make_async_copy(sd_hbm.at[pl.ds(ci * SD, SD)], sd_v,
                              sidx).wait()
        pltpu.make_async_copy(wp_hbm.at[pl.ds(ci * WPC, WPC)], wp_v,
                              sidx).wait()

    def scale(rows_v, wp_v):
        def sc(e, _):
            wrow = wp_v[pl.ds(e * LANES, LANES)]
            for k in range(HHID // LANES):
                rows_v[e, pl.ds(k * LANES, LANES)] = (
                    rows_v[e, pl.ds(k * LANES, LANES)] * wrow)
            return 0
        lax.fori_loop(0, C, sc, 0, unroll=4)

    bufs = ((sd0, wp0, src0, dst0, rows0), (sd1, wp1, src1, dst1, rows1))

    pltpu.sync_copy(sd_hbm.at[pl.ds(g0 * SD, SD)], sd0)
    pltpu.sync_copy(wp_hbm.at[pl.ds(g0 * WPC, WPC)], wp0)
    build(sd0, src0, dst0)
    pltpu.async_copy(h_hbm.at[src0], rows0, sgat)
    issue_idx(g0 + 1, sd1, wp1)

    def sub(ci, A, B):
        sdA, wpA, srcA, dstA, rowsA = A
        sdB, wpB, srcB, dstB, rowsB = B
        pltpu.make_async_copy(h_hbm.at[srcA], rowsA, sgat).wait()

        @pl.when(ci > g0)
        def _():
            pltpu.make_async_copy(rowsB, msg_sh.at[dstB], ssc).wait()
        wait_idx(ci + 1, sdB, wpB)
        build(sdB, srcB, dstB)
        pltpu.async_copy(h_hbm.at[srcB], rowsB, sgat)
        scale(rowsA, wpA)

        @pl.when(ci + 2 < gend)
        def _():
            issue_idx(ci + 2, sdA, wpA)
        pltpu.async_copy(rowsA, msg_sh.at[dstA], ssc, add=True)

    def body(i, _):
        ci0 = g0 + i * 2
        sub(ci0, bufs[0], bufs[1])
        sub(ci0 + 1, bufs[1], bufs[0])
        return 0
    lax.fori_loop(0, (NCH - 1) // 2, body, 0)

    lb = (NCH - 1) % 2
    _, wpL, srcL, dstL, rowsL = bufs[lb]
    sdP, wpP, srcP, dstP, rowsP = bufs[1 - lb]
    pltpu.make_async_copy(h_hbm.at[srcL], rowsL, sgat).wait()
    pltpu.make_async_copy(rowsP, msg_sh.at[dstP], ssc).wait()
    scale(rowsL, wpL)
    pltpu.sync_copy(rowsL, msg_sh.at[dstL], add=True)

    plsc.subcore_barrier()
    pltpu.sync_copy(msg_sh.at[pl.ds(s * RPT, RPT)],
                    msg_hbm.at[pl.ds(c * NP + s * RPT, RPT)])


def _msg(h2flat, sd, wp, N, NP):
    E = sd.shape[0] // 2
    C = CEDGE
    EPT = E // NS
    RPT = NP // NS
    assert (EPT // C) % 2 == 1 and EPT % C == 0
    body = functools.partial(_msg_body, N=N, NP=NP, C=C, EPT=EPT, RPT=RPT)
    f = pl.kernel(
        body,
        out_type=jax.ShapeDtypeStruct((NC * NP, HHID), jnp.float32),
        mesh=_mesh(),
        scratch_types=[
            pltpu.VMEM((2 * C,), jnp.int32),
            pltpu.VMEM((2 * C,), jnp.int32),
            pltpu.VMEM((C * LANES,), jnp.float32),
            pltpu.VMEM((C * LANES,), jnp.float32),
            pltpu.VMEM((C,), jnp.int32),
            pltpu.VMEM((C,), jnp.int32),
            pltpu.VMEM((C,), jnp.int32),
            pltpu.VMEM((C,), jnp.int32),
            pltpu.VMEM((C, HHID), jnp.float32),
            pltpu.VMEM((C, HHID), jnp.float32),
            pltpu.VMEM_SHARED((NP, HHID), jnp.float32),
            pltpu.SemaphoreType.DMA,
            pltpu.SemaphoreType.DMA,
            pltpu.SemaphoreType.DMA,
        ],
    )
    return f(h2flat, sd, wp)


def _proj_split_body(x_ref, w_ref, b_ref, o_ref):
    y = jnp.dot(x_ref[...], w_ref[...],
                preferred_element_type=jnp.float32) + b_ref[...]
    o_ref[0] = y[:, :HHID]
    o_ref[1] = y[:, HHID:]


def _proj_split(x, W, b, BR=512):
    N, D = x.shape
    G = (N + BR - 1) // BR
    return pl.pallas_call(
        _proj_split_body,
        grid=(G,),
        in_specs=[
            pl.BlockSpec((BR, D), lambda i: (i, 0)),
            pl.BlockSpec((D, HID), lambda i: (0, 0)),
            pl.BlockSpec((1, HID), lambda i: (0, 0)),
        ],
        out_specs=pl.BlockSpec((NC, BR, HHID), lambda i: (0, i, 0)),
        out_shape=jax.ShapeDtypeStruct((NC, N, HHID), jnp.float32),
    )(x, W, b.reshape(1, HID))


def _gru_core(h_ref, m_ref, deg_ref, wz_ref, wr_ref, wh_ref, bz_ref, br_ref,
              bh_ref, g_ref, bt_ref):
    h = jnp.concatenate([h_ref[0], h_ref[1]], axis=-1)
    m = jnp.concatenate([m_ref[0], m_ref[1]], axis=-1)
    m = m / jnp.maximum(deg_ref[...][:, :1], 1.0)
    hm = jnp.concatenate([h, m], axis=-1)
    z = jax.nn.sigmoid(jnp.dot(hm, wz_ref[...],
                               preferred_element_type=jnp.float32)
                       + bz_ref[...])
    r = jax.nn.sigmoid(jnp.dot(hm, wr_ref[...],
                               preferred_element_type=jnp.float32)
                       + br_ref[...])
    hr = jnp.concatenate([r * h, m], axis=-1)
    ht = jnp.tanh(jnp.dot(hr, wh_ref[...],
                          preferred_element_type=jnp.float32) + bh_ref[...])
    hn = (1.0 - z) * h + z * ht
    mu = jnp.mean(hn, axis=-1, keepdims=True)
    var = jnp.mean((hn - mu) ** 2, axis=-1, keepdims=True)
    return (hn - mu) / jnp.sqrt(var + EPS_LN) * g_ref[...] + bt_ref[...]


def _gru_body(h_ref, m_ref, deg_ref, wz_ref, wr_ref, wh_ref, bz_ref, br_ref,
              bh_ref, g_ref, bt_ref, o_ref):
    y = _gru_core(h_ref, m_ref, deg_ref, wz_ref, wr_ref, wh_ref, bz_ref,
                  br_ref, bh_ref, g_ref, bt_ref)
    o_ref[0] = y[:, :HHID]
    o_ref[1] = y[:, HHID:]


def _gru_out_body(h_ref, m_ref, deg_ref, wz_ref, wr_ref, wh_ref, bz_ref,
                  br_ref, bh_ref, g_ref, bt_ref, wo_ref, bo_ref, o_ref):
    y = _gru_core(h_ref, m_ref, deg_ref, wz_ref, wr_ref, wh_ref, bz_ref,
                  br_ref, bh_ref, g_ref, bt_ref)
    o_ref[...] = jnp.dot(y, wo_ref[...],
                         preferred_element_type=jnp.float32) + bo_ref[...]


def _gru(h2, msg2, deg, lp, Wo=None, bo=None, BR=512):
    N = h2.shape[1]
    G = (N + BR - 1) // BR
    spec_w = pl.BlockSpec((2 * HID, HID), lambda i: (0, 0))
    spec_b = pl.BlockSpec((1, HID), lambda i: (0, 0))
    in_specs = [
        pl.BlockSpec((NC, BR, HHID), lambda i: (0, i, 0)),
        pl.BlockSpec((NC, BR, HHID), lambda i: (0, i, 0)),
        pl.BlockSpec((BR, HHID), lambda i: (i, 0)),
        spec_w, spec_w, spec_w, spec_b, spec_b, spec_b, spec_b, spec_b,
    ]
    args = [h2, msg2, deg, lp['Wz'], lp['Wr'], lp['Wh'],
            lp['bz'].reshape(1, HID), lp['br'].reshape(1, HID),
            lp['bh'].reshape(1, HID), lp['g'].reshape(1, HID),
            lp['bt'].reshape(1, HID)]
    if Wo is None:
        return pl.pallas_call(
            _gru_body,
            grid=(G,),
            in_specs=in_specs,
            out_specs=pl.BlockSpec((NC, BR, HHID), lambda i: (0, i, 0)),
            out_shape=jax.ShapeDtypeStruct((NC, N, HHID), jnp.float32),
        )(*args)
    in_specs += [pl.BlockSpec((HID, HID), lambda i: (0, 0)),
                 pl.BlockSpec((1, HID), lambda i: (0, 0))]
    args += [Wo, bo.reshape(1, HID)]
    return pl.pallas_call(
        _gru_out_body,
        grid=(G,),
        in_specs=in_specs,
        out_specs=pl.BlockSpec((BR, HID), lambda i: (i, 0)),
        out_shape=jax.ShapeDtypeStruct((N, HID), jnp.float32),
    )(*args)


def kernel(x, edge_index, path_lengths, merge_counts, params):
    N, D = x.shape
    E = edge_index.shape[1]
    src = edge_index[0].astype(jnp.int32)
    dst = edge_index[1].astype(jnp.int32)

    scale16 = jnp.broadcast_to(-1.0 / (params['tau'] + EPS_TAU),
                               (LANES,)).astype(jnp.float32)
    pw = jnp.concatenate([path_lengths.reshape(-1, CEDGE),
                          merge_counts.reshape(-1, CEDGE)],
                         axis=1).reshape(-1)
    deg_flat, wp, NP = _degwp(dst, pw, scale16, N)
    deg = deg_flat[:NP]
    sd = jnp.concatenate([src.reshape(-1, CEDGE), dst.reshape(-1, CEDGE)],
                         axis=1).reshape(-1)

    h2 = _proj_split(x, params['W_in'], params['b_in'])
    nl = len(params['layers'])
    for li, lp in enumerate(params['layers']):
        msg_flat = _msg(h2.reshape(NC * N, HHID), sd, wp, N, NP)
        msg2 = msg_flat.reshape(NC, NP, HHID)
        if li < nl - 1:
            h2 = _gru(h2, msg2, deg, lp)
        else:
            return _gru(h2, msg2, deg, lp,
                        Wo=params['W_out'], bo=params['b_out'])


# speedup vs baseline: 2.4052x; 1.0996x over previous
import functools
import numpy as np
import jax
import jax.numpy as jnp
from jax import lax
from jax.experimental import pallas as pl
from jax.experimental.pallas import tpu as pltpu
from jax.experimental.pallas import tpu_sc as plsc

HID = 256
NL = 3
DECAY = 0.9
EPS_TAU = 1e-08
EPS_LN = 1e-05
LN_DECAY = float(np.log(DECAY))

NC = 2
NS = 16
LANES = 16
HHID = HID // 2
CEDGE = 80


def _mesh():
    return plsc.VectorSubcoreMesh(core_axis_name="c", subcore_axis_name="s")


def _degwp_body(dst_hbm, pw_hbm, scale_hbm, deg_hbm, wp_hbm,
                dst0, dst1, pw0, pw1, wq16_v, ones_v, scale_v, deg_sh, sidx,
                *, E, NP, C, EPT, RPT):
    c = lax.axis_index("c")
    s = lax.axis_index("s")
    NCH = EPT // C
    PW = 2 * C

    def fill(val, i, _):
        for k in range(HHID // LANES):
            ones_v[i, pl.ds(k * LANES, LANES)] = jnp.full((LANES,), val,
                                                          jnp.float32)
        return 0
    lax.fori_loop(0, C, functools.partial(fill, 0.0), 0)
    zbase = s * RPT
    for t in range(RPT // C):
        pltpu.sync_copy(ones_v, deg_sh.at[pl.ds(zbase + t * C, C)])
    if RPT % C:
        pltpu.sync_copy(ones_v.at[pl.ds(0, RPT % C)],
                        deg_sh.at[pl.ds(zbase + (RPT // C) * C, RPT % C)])
    lax.fori_loop(0, C, functools.partial(fill, 1.0), 0)
    pltpu.sync_copy(scale_hbm, scale_v)
    plsc.subcore_barrier()

    g0 = s * NCH
    gend = g0 + NCH

    def issue_idx(ci, dst_v, pw_v):
        pltpu.async_copy(dst_hbm.at[pl.ds(ci * C, C)], dst_v, sidx)
        pltpu.async_copy(pw_hbm.at[pl.ds(ci * PW, PW)], pw_v, sidx)

    def wait_idx(ci, dst_v, pw_v):
        pltpu.make_async_copy(dst_hbm.at[pl.ds(ci * C, C)], dst_v,
                              sidx).wait()
        pltpu.make_async_copy(pw_hbm.at[pl.ds(ci * PW, PW)], pw_v,
                              sidx).wait()

    def wq(ci, pw_v):
        @pl.when(c == 0)
        def _():
            for j in range(C // LANES):
                pv = pw_v[pl.ds(j * LANES, LANES)]
                mv = pw_v[pl.ds(C + j * LANES, LANES)]
                dist = pv * jnp.exp(mv * LN_DECAY)
                w = jnp.exp(dist * scale_v[...])
                for kk in range(LANES):
                    e = j * LANES + kk
                    wq16_v[pl.ds(e * LANES, LANES)] = (
                        jnp.broadcast_to(w[kk], (LANES,)))
            pltpu.sync_copy(wq16_v,
                            wp_hbm.at[pl.ds(ci * C * LANES, C * LANES)])

    bufs = ((dst0, pw0), (dst1, pw1))
    pltpu.sync_copy(dst_hbm.at[pl.ds(g0 * C, C)], dst0)
    pltpu.sync_copy(pw_hbm.at[pl.ds(g0 * PW, PW)], pw0)
    issue_idx(g0 + 1, dst1, pw1)

    def sub(ci, A, B):
        dstA, pwA = A
        dstB, pwB = B
        wq(ci, pwA)
        wait_idx(ci + 1, dstB, pwB)

        @pl.when(ci + 2 < gend)
        def _():
            issue_idx(ci + 2, dstA, pwA)
        pltpu.sync_copy(ones_v, deg_sh.at[dstA], add=True)

    def body(i, _):
        ci0 = g0 + i * 2
        sub(ci0, bufs[0], bufs[1])
        sub(ci0 + 1, bufs[1], bufs[0])
        return 0
    lax.fori_loop(0, (NCH - 1) // 2, body, 0)

    lb = (NCH - 1) % 2
    dstL, pwL = bufs[lb]
    wq(gend - 1, pwL)
    pltpu.sync_copy(ones_v, deg_sh.at[dstL], add=True)

    plsc.subcore_barrier()
    pltpu.sync_copy(deg_sh.at[pl.ds(s * RPT, RPT)],
                    deg_hbm.at[pl.ds(c * NP + s * RPT, RPT)])


def _degwp(dst, pw, scale16, N):
    E = dst.shape[0]
    NP = ((N + NS * 8 - 1) // (NS * 8)) * (NS * 8)
    C = CEDGE
    EPT = E // NS
    RPT = NP // NS
    assert (EPT // C) % 2 == 1 and EPT % C == 0
    body = functools.partial(_degwp_body, E=E, NP=NP, C=C, EPT=EPT, RPT=RPT)
    f = pl.kernel(
        body,
        out_type=(jax.ShapeDtypeStruct((NC * NP, HHID), jnp.float32),
                  jax.ShapeDtypeStruct((E * LANES,), jnp.float32)),
        mesh=_mesh(),
        scratch_types=[
            pltpu.VMEM((C,), jnp.int32),
            pltpu.VMEM((C,), jnp.int32),
            pltpu.VMEM((2 * C,), jnp.float32),
            pltpu.VMEM((2 * C,), jnp.float32),
            pltpu.VMEM((C * LANES,), jnp.float32),
            pltpu.VMEM((C, HHID), jnp.float32),
            pltpu.VMEM((LANES,), jnp.float32),
            pltpu.VMEM_SHARED((NP, HHID), jnp.float32),
            pltpu.SemaphoreType.DMA,
        ],
    )
    return f(dst, pw, scale16) + (NP,)


def _msg_body(h_hbm, sd_hbm, wp_hbm, msg_hbm,
              sd0, sd1, sd2, wp0, wp1, wp2, src0, src1, src2,
              dst0, dst1, dst2, rows0, rows1, rows2,
              msg_sh, sidx, sgat, ssc,
              *, N, NP, C, EPT, RPT):
    c = lax.axis_index("c")
    s = lax.axis_index("s")
    NCH = EPT // C
    SD = 2 * C
    WPC = C * LANES

    def fill_zero(i, _):
        for k in range(HHID // LANES):
            rows0[i, pl.ds(k * LANES, LANES)] = jnp.zeros((LANES,),
                                                          jnp.float32)
        return 0
    lax.fori_loop(0, C, fill_zero, 0)
    zbase = s * RPT
    for t in range(RPT // C):
        pltpu.sync_copy(rows0, msg_sh.at[pl.ds(zbase + t * C, C)])
    if RPT % C:
        pltpu.sync_copy(rows0.at[pl.ds(0, RPT % C)],
                        msg_sh.at[pl.ds(zbase + (RPT // C) * C, RPT % C)])
    plsc.subcore_barrier()

    g0 = s * NCH
    gend = g0 + NCH
    coff = c * N

    def build(sd_v, src_v, dst_v):
        for j in range(C // LANES):
            src_v[pl.ds(j * LANES, LANES)] = (
                sd_v[pl.ds(j * LANES, LANES)] + coff)
            dst_v[pl.ds(j * LANES, LANES)] = sd_v[pl.ds(C + j * LANES, LANES)]

    def issue_idx(ci, sd_v, wp_v):
        pltpu.async_copy(sd_hbm.at[pl.ds(ci * SD, SD)], sd_v, sidx)
        pltpu.async_copy(wp_hbm.at[pl.ds(ci * WPC, WPC)], wp_v, sidx)

    def wait_idx(ci, sd_v, wp_v):
        pltpu.make_async_copy(sd_hbm.at[pl.ds(ci * SD, SD)], sd_v,
                              sidx).wait()
        pltpu.make_async_copy(wp_hbm.at[pl.ds(ci * WPC, WPC)], wp_v,
                              sidx).wait()

    def scale(rows_v, wp_v):
        def sc(e, _):
            wrow = wp_v[pl.ds(e * LANES, LANES)]
            for k in range(HHID // LANES):
                rows_v[e, pl.ds(k * LANES, LANES)] = (
                    rows_v[e, pl.ds(k * LANES, LANES)] * wrow)
            return 0
        lax.fori_loop(0, C, sc, 0, unroll=4)

    bufs = ((sd0, wp0, src0, dst0, rows0), (sd1, wp1, src1, dst1, rows1),
            (sd2, wp2, src2, dst2, rows2))

    pltpu.sync_copy(sd_hbm.at[pl.ds(g0 * SD, SD)], sd0)
    pltpu.sync_copy(wp_hbm.at[pl.ds(g0 * WPC, WPC)], wp0)
    build(sd0, src0, dst0)
    pltpu.async_copy(h_hbm.at[src0], rows0, sgat)
    issue_idx(g0 + 1, sd1, wp1)
    issue_idx(g0 + 2, sd2, wp2)

    def sub(ci, A, B):
        sdA, wpA, srcA, dstA, rowsA = A
        sdB, wpB, srcB, dstB, rowsB = B
        pltpu.make_async_copy(h_hbm.at[srcA], rowsA, sgat).wait()

        @pl.when(ci >= g0 + 2)
        def _():
            pltpu.make_async_copy(rowsB, msg_sh.at[dstB], ssc).wait()
        wait_idx(ci + 1, sdB, wpB)
        build(sdB, srcB, dstB)
        pltpu.async_copy(h_hbm.at[srcB], rowsB, sgat)
        scale(rowsA, wpA)

        @pl.when(ci + 3 < gend)
        def _():
            issue_idx(ci + 3, sdA, wpA)
        pltpu.async_copy(rowsA, msg_sh.at[dstA], ssc, add=True)

    def body(i, _):
        ci0 = g0 + i * 3
        sub(ci0, bufs[0], bufs[1])
        sub(ci0 + 1, bufs[1], bufs[2])
        sub(ci0 + 2, bufs[2], bufs[0])
        return 0
    lax.fori_loop(0, (NCH - 2) // 3, body, 0)
    b1 = (NCH - 2) % 3
    sub(gend - 2, bufs[b1], bufs[(b1 + 1) % 3])

    lb = (NCH - 1) % 3
    _, wpL, srcL, dstL, rowsL = bufs[lb]
    _, _, _, dstP2, rowsP2 = bufs[(lb + 1) % 3]
    _, _, _, dstP1, rowsP1 = bufs[(lb + 2) % 3]
    pltpu.make_async_copy(h_hbm.at[srcL], rowsL, sgat).wait()
    pltpu.make_async_copy(rowsP2, msg_sh.at[dstP2], ssc).wait()
    scale(rowsL, wpL)
    pltpu.async_copy(rowsL, msg_sh.at[dstL], ssc, add=True)
    pltpu.make_async_copy(rowsP1, msg_sh.at[dstP1], ssc).wait()
    pltpu.make_async_copy(rowsL, msg_sh.at[dstL], ssc).wait()

    plsc.subcore_barrier()
    pltpu.sync_copy(msg_sh.at[pl.ds(s * RPT, RPT)],
                    msg_hbm.at[pl.ds(c * NP + s * RPT, RPT)])


def _msg(h2flat, sd, wp, N, NP):
    E = sd.shape[0] // 2
    C = CEDGE
    EPT = E // NS
    RPT = NP // NS
    assert EPT % C == 0 and (EPT // C - 2) % 3 == 0 and EPT // C >= 5
    body = functools.partial(_msg_body, N=N, NP=NP, C=C, EPT=EPT, RPT=RPT)
    f = pl.kernel(
        body,
        out_type=jax.ShapeDtypeStruct((NC * NP, HHID), jnp.float32),
        mesh=_mesh(),
        scratch_types=(
            [pltpu.VMEM((2 * C,), jnp.int32)] * 3
            + [pltpu.VMEM((C * LANES,), jnp.float32)] * 3
            + [pltpu.VMEM((C,), jnp.int32)] * 6
            + [pltpu.VMEM((C, HHID), jnp.float32)] * 3
            + [pltpu.VMEM_SHARED((NP, HHID), jnp.float32),
               pltpu.SemaphoreType.DMA,
               pltpu.SemaphoreType.DMA,
               pltpu.SemaphoreType.DMA]
        ),
    )
    return f(h2flat, sd, wp)


def _proj_split_body(x_ref, w_ref, b_ref, o_ref):
    y = jnp.dot(x_ref[...], w_ref[...],
                preferred_element_type=jnp.float32) + b_ref[...]
    o_ref[0] = y[:, :HHID]
    o_ref[1] = y[:, HHID:]


def _proj_split(x, W, b, BR=512):
    N, D = x.shape
    G = (N + BR - 1) // BR
    return pl.pallas_call(
        _proj_split_body,
        grid=(G,),
        in_specs=[
            pl.BlockSpec((BR, D), lambda i: (i, 0)),
            pl.BlockSpec((D, HID), lambda i: (0, 0)),
            pl.BlockSpec((1, HID), lambda i: (0, 0)),
        ],
        out_specs=pl.BlockSpec((NC, BR, HHID), lambda i: (0, i, 0)),
        out_shape=jax.ShapeDtypeStruct((NC, N, HHID), jnp.float32),
    )(x, W, b.reshape(1, HID))


def _gru_core(h_ref, m_ref, deg_ref, wz_ref, wr_ref, wh_ref, bz_ref, br_ref,
              bh_ref, g_ref, bt_ref):
    h = jnp.concatenate([h_ref[0], h_ref[1]], axis=-1)
    m = jnp.concatenate([m_ref[0], m_ref[1]], axis=-1)
    m = m / jnp.maximum(deg_ref[...][:, :1], 1.0)
    hm = jnp.concatenate([h, m], axis=-1)
    z = jax.nn.sigmoid(jnp.dot(hm, wz_ref[...],
                               preferred_element_type=jnp.float32)
                       + bz_ref[...])
    r = jax.nn.sigmoid(jnp.dot(hm, wr_ref[...],
                               preferred_element_type=jnp.float32)
                       + br_ref[...])
    hr = jnp.concatenate([r * h, m], axis=-1)
    ht = jnp.tanh(jnp.dot(hr, wh_ref[...],
                          preferred_element_type=jnp.float32) + bh_ref[...])
    hn = (1.0 - z) * h + z * ht
    mu = jnp.mean(hn, axis=-1, keepdims=True)
    var = jnp.mean((hn - mu) ** 2, axis=-1, keepdims=True)
    return (hn - mu) / jnp.sqrt(var + EPS_LN) * g_ref[...] + bt_ref[...]


def _gru_body(h_ref, m_ref, deg_ref, wz_ref, wr_ref, wh_ref, bz_ref, br_ref,
              bh_ref, g_ref, bt_ref, o_ref):
    y = _gru_core(h_ref, m_ref, deg_ref, wz_ref, wr_ref, wh_ref, bz_ref,
                  br_ref, bh_ref, g_ref, bt_ref)
    o_ref[0] = y[:, :HHID]
    o_ref[1] = y[:, HHID:]


def _gru_out_body(h_ref, m_ref, deg_ref, wz_ref, wr_ref, wh_ref, bz_ref,
                  br_ref, bh_ref, g_ref, bt_ref, wo_ref, bo_ref, o_ref):
    y = _gru_core(h_ref, m_ref, deg_ref, wz_ref, wr_ref, wh_ref, bz_ref,
                  br_ref, bh_ref, g_ref, bt_ref)
    o_ref[...] = jnp.dot(y, wo_ref[...],
                         preferred_element_type=jnp.float32) + bo_ref[...]


def _gru(h2, msg2, deg, lp, Wo=None, bo=None, BR=512):
    N = h2.shape[1]
    G = (N + BR - 1) // BR
    spec_w = pl.BlockSpec((2 * HID, HID), lambda i: (0, 0))
    spec_b = pl.BlockSpec((1, HID), lambda i: (0, 0))
    in_specs = [
        pl.BlockSpec((NC, BR, HHID), lambda i: (0, i, 0)),
        pl.BlockSpec((NC, BR, HHID), lambda i: (0, i, 0)),
        pl.BlockSpec((BR, HHID), lambda i: (i, 0)),
        spec_w, spec_w, spec_w, spec_b, spec_b, spec_b, spec_b, spec_b,
    ]
    args = [h2, msg2, deg, lp['Wz'], lp['Wr'], lp['Wh'],
            lp['bz'].reshape(1, HID), lp['br'].reshape(1, HID),
            lp['bh'].reshape(1, HID), lp['g'].reshape(1, HID),
            lp['bt'].reshape(1, HID)]
    if Wo is None:
        return pl.pallas_call(
            _gru_body,
            grid=(G,),
            in_specs=in_specs,
            out_specs=pl.BlockSpec((NC, BR, HHID), lambda i: (0, i, 0)),
            out_shape=jax.ShapeDtypeStruct((NC, N, HHID), jnp.float32),
        )(*args)
    in_specs += [pl.BlockSpec((HID, HID), lambda i: (0, 0)),
                 pl.BlockSpec((1, HID), lambda i: (0, 0))]
    args += [Wo, bo.reshape(1, HID)]
    return pl.pallas_call(
        _gru_out_body,
        grid=(G,),
        in_specs=in_specs,
        out_specs=pl.BlockSpec((BR, HID), lambda i: (i, 0)),
        out_shape=jax.ShapeDtypeStruct((N, HID), jnp.float32),
    )(*args)


def kernel(x, edge_index, path_lengths, merge_counts, params):
    N, D = x.shape
    E = edge_index.shape[1]
    src = edge_index[0].astype(jnp.int32)
    dst = edge_index[1].astype(jnp.int32)

    scale16 = jnp.broadcast_to(-1.0 / (params['tau'] + EPS_TAU),
                               (LANES,)).astype(jnp.float32)
    pw = jnp.concatenate([path_lengths.reshape(-1, CEDGE),
                          merge_counts.reshape(-1, CEDGE)],
                         axis=1).reshape(-1)
    deg_flat, wp, NP = _degwp(dst, pw, scale16, N)
    deg = deg_flat[:NP]
    sd = jnp.concatenate([src.reshape(-1, CEDGE), dst.reshape(-1, CEDGE)],
                         axis=1).reshape(-1)

    h2 = _proj_split(x, params['W_in'], params['b_in'])
    nl = len(params['layers'])
    for li, lp in enumerate(params['layers']):
        msg_flat = _msg(h2.reshape(NC * N, HHID), sd, wp, N, NP)
        msg2 = msg_flat.reshape(NC, NP, HHID)
        if li < nl - 1:
            h2 = _gru(h2, msg2, deg, lp)
        else:
            return _gru(h2, msg2, deg, lp,
                        Wo=params['W_out'], bo=params['b_out'])
